# baseline (speedup 1.0000x reference)
import functools

import jax
import jax.numpy as jnp
from jax import lax
from jax.experimental import pallas as pl
from jax.experimental.pallas import tpu as pltpu
from jax.experimental.pallas import tpu_sc as plsc

B, S, DIM = 1, 2048, 768
E, TOPK, HID = 8, 2, 2048

NA = S * TOPK
BM = 128
NB = NA // BM + E
NPAD = NB * BM
NBPAD = 64

NW = 32
CHUNK = NA // NW
TPW = S // NW
L = 16


def _router_body(x_ref, wg_ref, eid_ref, wt_ref):
    logits = jnp.dot(x_ref[...], wg_ref[...], preferred_element_type=jnp.float32)
    m = jnp.max(logits, axis=-1, keepdims=True)
    p = jnp.exp(logits - m)
    p = p / jnp.sum(p, axis=-1, keepdims=True)
    idx = lax.broadcasted_iota(jnp.int32, p.shape, 1)
    m1 = jnp.max(p, axis=-1, keepdims=True)
    i1 = jnp.min(jnp.where(p == m1, idx, E), axis=-1, keepdims=True)
    p2 = jnp.where(idx == i1, -jnp.inf, p)
    m2 = jnp.max(p2, axis=-1, keepdims=True)
    i2 = jnp.min(jnp.where(p2 == m2, idx, E), axis=-1, keepdims=True)
    denom = m1 + m2 + 1e-9
    eid_ref[...] = jnp.concatenate([i1, i2], axis=1)
    wt_ref[...] = jnp.concatenate([m1 / denom, m2 / denom], axis=1)


def _router(x2, wg):
    return pl.pallas_call(
        _router_body,
        out_shape=[
            jax.ShapeDtypeStruct((S, TOPK), jnp.int32),
            jax.ShapeDtypeStruct((S, TOPK), jnp.float32),
        ],
    )(x2, wg)


def _dispatch_body(eid_hbm, x_hbm, xs_hbm, dst_hbm, be_hbm,
                   eids_v, dst_v, tok_v, rows_v, be_v, sem):
    cid = lax.axis_index("c")
    sid = lax.axis_index("s")
    wid = sid * 2 + cid
    base = wid * CHUNK
    first_vreg = wid * (CHUNK // L)

    pltpu.sync_copy(eid_hbm, eids_v)

    zero = jnp.zeros((L,), jnp.int32)

    def scan_body(r, accs):
        tot, pre = accs
        ev = eids_v[pl.ds(r * L, L)]
        pred = jnp.where(r < first_vreg, 1, 0).astype(jnp.int32)
        new_tot = []
        new_pre = []
        for e in range(E):
            c = (ev == e).astype(jnp.int32)
            new_tot.append(tot[e] + c)
            new_pre.append(pre[e] + c * pred)
        return (tuple(new_tot), tuple(new_pre))

    tot, pre = lax.fori_loop(
        0, NA // L, scan_body,
        (tuple(zero for _ in range(E)), tuple(zero for _ in range(E))),
    )
    cnt = [jnp.sum(tot[e]) for e in range(E)]
    pre_run = [jnp.sum(pre[e]) for e in range(E)]

    off = []
    o = jnp.int32(0)
    cumblk = []
    for e in range(E):
        off.append(o)
        nb = (cnt[e] + (BM - 1)) // BM
        o = o + nb * BM
        cumblk.append(o // BM)

    for j in range(CHUNK // L):
        ev = eids_v[pl.ds((first_vreg + j) * L, L)]
        dstv = jnp.zeros((L,), jnp.int32)
        for e in range(E):
            mi = (ev == e).astype(jnp.int32)
            rank = jnp.cumsum(mi) - 1
            dstv = dstv + mi * (off[e] + pre_run[e] + rank)
            pre_run[e] = pre_run[e] + jnp.sum(mi)
        dst_v[pl.ds(j * L, L)] = dstv
        a_iota = lax.broadcasted_iota(jnp.int32, (L,), 0)
        tok_v[pl.ds(j * L, L)] = (base + j * L + a_iota) // TOPK

    pltpu.sync_copy(dst_v, dst_hbm.at[pl.ds(base, CHUNK)])
    pltpu.async_copy(x_hbm.at[tok_v], rows_v, sem).wait()
    pltpu.async_copy(rows_v, xs_hbm.at[dst_v], sem).wait()

    @pl.when(wid == 0)
    def _():
        e_last = jnp.int32(0)
        for e in range(1, E):
            e_last = jnp.where(cnt[e] > 0, jnp.int32(e), e_last)
        tb = cumblk[E - 1]
        for k in range(NBPAD // L):
            bv = lax.broadcasted_iota(jnp.int32, (L,), 0) + k * L
            bev = jnp.zeros((L,), jnp.int32)
            for e in range(E - 1):
                bev = bev + (bv >= cumblk[e]).astype(jnp.int32)
            bev = jnp.minimum(bev, e_last)
            bev = jnp.where(bv == NB, tb, bev)
            be_v[pl.ds(k * L, L)] = bev
        pltpu.sync_copy(be_v, be_hbm)


def _dispatch(eflat, x2):
    return pl.kernel(
        _dispatch_body,
        mesh=plsc.VectorSubcoreMesh(core_axis_name="c", subcore_axis_name="s"),
        compiler_params=pltpu.CompilerParams(needs_layout_passes=False),
        out_type=[
            jax.ShapeDtypeStruct((NPAD, DIM), jnp.float32),
            jax.ShapeDtypeStruct((NA,), jnp.int32),
            jax.ShapeDtypeStruct((NBPAD,), jnp.int32),
        ],
        scratch_types=[
            pltpu.VMEM((NA,), jnp.int32),
            pltpu.VMEM((CHUNK,), jnp.int32),
            pltpu.VMEM((CHUNK,), jnp.int32),
            pltpu.VMEM((CHUNK, DIM), jnp.float32),
            pltpu.VMEM((NBPAD,), jnp.int32),
            pltpu.SemaphoreType.DMA,
        ],
    )(eflat, x2)


CH = 512
NC = HID // CH


def _gemm_body(be_ref, xs_ref, w1_ref, b1_ref, w2_ref, b2_ref, ys_ref):
    c = pl.program_id(0)
    i = pl.program_id(1)

    @pl.when(i < be_ref[NB])
    def _():
        xb = xs_ref[pl.ds(i * BM, BM), :]
        h = jnp.maximum(
            jnp.dot(xb, w1_ref[0], preferred_element_type=jnp.float32)
            + b1_ref[0],
            0.0,
        )
        partial = jnp.dot(h, w2_ref[0], preferred_element_type=jnp.float32)

        @pl.when(c == 0)
        def _():
            ys_ref[pl.ds(i * BM, BM), :] = partial + b2_ref[0]

        @pl.when(c != 0)
        def _():
            ys_ref[pl.ds(i * BM, BM), :] = ys_ref[pl.ds(i * BM, BM), :] + partial


def _gemm(be, xs, W1, b1, W2, b2):
    grid_spec = pltpu.PrefetchScalarGridSpec(
        num_scalar_prefetch=1,
        grid=(NC, NB),
        in_specs=[
            pl.BlockSpec((NPAD, DIM), lambda c, i, be_ref: (0, 0)),
            pl.BlockSpec((1, DIM, CH), lambda c, i, be_ref: (be_ref[i], 0, c)),
            pl.BlockSpec((1, 1, CH), lambda c, i, be_ref: (be_ref[i], 0, c)),
            pl.BlockSpec((1, CH, DIM), lambda c, i, be_ref: (be_ref[i], c, 0)),
            pl.BlockSpec((1, 1, DIM), lambda c, i, be_ref: (0, 0, 0)),
        ],
        out_specs=pl.BlockSpec((NPAD, DIM), lambda c, i, be_ref: (0, 0)),
    )
    return pl.pallas_call(
        _gemm_body,
        grid_spec=grid_spec,
        out_shape=jax.ShapeDtypeStruct((NPAD, DIM), jnp.float32),
        compiler_params=pltpu.CompilerParams(
            dimension_semantics=("arbitrary", "arbitrary"),
        ),
    )(be, xs, W1, b1.reshape(E, 1, HID), W2, b2.reshape(E, 1, DIM))


CB = 4
CBA = CHUNK // CB
CBT = TPW // CB


def _combine_body(ys_hbm, dst_hbm, w_hbm, y_hbm,
                  dst_v, w_v, rows_a, rows_b, out_v, sem_a, sem_b):
    cid = lax.axis_index("c")
    sid = lax.axis_index("s")
    wid = sid * 2 + cid
    abase = wid * CHUNK
    tbase = wid * TPW

    pltpu.sync_copy(dst_hbm.at[pl.ds(abase, CHUNK)], dst_v)
    pltpu.sync_copy(w_hbm.at[pl.ds(abase, CHUNK)], w_v)

    bufs = [(rows_a, sem_a), (rows_b, sem_b)]

    def gather(bi):
        rows, sem = bufs[bi % 2]
        return pltpu.async_copy(
            ys_hbm.at[dst_v.at[pl.ds(bi * CBA, CBA)]], rows, sem
        )

    pending = gather(0)
    for bi in range(CB):
        rows_v, _ = bufs[bi % 2]
        pending.wait()
        if bi + 1 < CB:
            pending = gather(bi + 1)

        def tok_body(t, _):
            wa = plsc.load_gather(w_v, [jnp.full((L,), bi * CBA + 2 * t, jnp.int32)])
            wb = plsc.load_gather(w_v, [jnp.full((L,), bi * CBA + 2 * t + 1, jnp.int32)])
            for j in range(DIM // L):
                a = rows_v[2 * t, pl.ds(j * L, L)]
                b = rows_v[2 * t + 1, pl.ds(j * L, L)]
                out_v[t, pl.ds(j * L, L)] = wa * a + wb * b
            return 0

        lax.fori_loop(0, CBT, tok_body, 0)
        pltpu.sync_copy(out_v, y_hbm.at[pl.ds(tbase + bi * CBT, CBT)])


def _combine(ys, dst, wflat):
    return pl.kernel(
        _combine_body,
        mesh=plsc.VectorSubcoreMesh(core_axis_name="c", subcore_axis_name="s"),
        compiler_params=pltpu.CompilerParams(needs_layout_passes=False),
        out_type=jax.ShapeDtypeStruct((S, DIM), jnp.float32),
        scratch_types=[
            pltpu.VMEM((CHUNK,), jnp.int32),
            pltpu.VMEM((CHUNK,), jnp.float32),
            pltpu.VMEM((CBA, DIM), jnp.float32),
            pltpu.VMEM((CBA, DIM), jnp.float32),
            pltpu.VMEM((CBT, DIM), jnp.float32),
            pltpu.SemaphoreType.DMA,
            pltpu.SemaphoreType.DMA,
        ],
    )(ys, dst, wflat)


@jax.jit
def kernel(x, Wg, W1, b1, W2, b2):
    x2 = x.reshape(S, DIM)
    eid, wt = _router(x2, Wg)
    xs, dst, be = _dispatch(eid.reshape(NA), x2)
    ys = _gemm(be[: NB + 1], xs, W1, b1, W2, b2)
    y = _combine(ys, dst, wt.reshape(NA))
    return y.reshape(B, S, DIM)


# speedup vs baseline: 1.5318x; 1.5318x over previous
import functools

import jax
import jax.numpy as jnp
from jax import lax
from jax.experimental import pallas as pl
from jax.experimental.pallas import tpu as pltpu
from jax.experimental.pallas import tpu_sc as plsc

B, S, DIM = 1, 2048, 768
E, TOPK, HID = 8, 2, 2048

NA = S * TOPK
BM = 128
NB = NA // BM + E
NPAD = NB * BM
NBPAD = 256

NW = 32
CHUNK = NA // NW
TPW = S // NW
L = 16


def _router_body(x_ref, wg_ref, eid_ref, wt_ref):
    logits = jnp.dot(x_ref[...], wg_ref[...], preferred_element_type=jnp.float32)
    m = jnp.max(logits, axis=-1, keepdims=True)
    p = jnp.exp(logits - m)
    p = p / jnp.sum(p, axis=-1, keepdims=True)
    idx = lax.broadcasted_iota(jnp.int32, p.shape, 1)
    m1 = jnp.max(p, axis=-1, keepdims=True)
    i1 = jnp.min(jnp.where(p == m1, idx, E), axis=-1, keepdims=True)
    p2 = jnp.where(idx == i1, -jnp.inf, p)
    m2 = jnp.max(p2, axis=-1, keepdims=True)
    i2 = jnp.min(jnp.where(p2 == m2, idx, E), axis=-1, keepdims=True)
    denom = m1 + m2 + 1e-9
    eid_ref[...] = jnp.concatenate([i1, i2], axis=1)
    wt_ref[...] = jnp.concatenate([m1 / denom, m2 / denom], axis=1)


def _router(x2, wg):
    return pl.pallas_call(
        _router_body,
        out_shape=[
            jax.ShapeDtypeStruct((S, TOPK), jnp.int32),
            jax.ShapeDtypeStruct((S, TOPK), jnp.float32),
        ],
    )(x2, wg)


def _dispatch_body(eid_hbm, x_hbm, xs_hbm, dst_hbm, be_hbm,
                   eids_v, dst_v, tok_v, rows_v, be_v, sem):
    cid = lax.axis_index("c")
    sid = lax.axis_index("s")
    wid = sid * 2 + cid
    base = wid * CHUNK
    first_vreg = wid * (CHUNK // L)

    pltpu.sync_copy(eid_hbm, eids_v)

    zero = jnp.zeros((L,), jnp.int32)

    def scan_body(r, accs):
        tot, pre = accs
        ev = eids_v[pl.ds(r * L, L)]
        pred = jnp.where(r < first_vreg, 1, 0).astype(jnp.int32)
        new_tot = []
        new_pre = []
        for e in range(E):
            c = (ev == e).astype(jnp.int32)
            new_tot.append(tot[e] + c)
            new_pre.append(pre[e] + c * pred)
        return (tuple(new_tot), tuple(new_pre))

    tot, pre = lax.fori_loop(
        0, NA // L, scan_body,
        (tuple(zero for _ in range(E)), tuple(zero for _ in range(E))),
    )
    cnt = [jnp.sum(tot[e]) for e in range(E)]
    pre_run = [jnp.sum(pre[e]) for e in range(E)]

    off = []
    o = jnp.int32(0)
    cumblk = []
    for e in range(E):
        off.append(o)
        nb = (cnt[e] + (BM - 1)) // BM
        o = o + nb * BM
        cumblk.append(o // BM)

    for j in range(CHUNK // L):
        ev = eids_v[pl.ds((first_vreg + j) * L, L)]
        dstv = jnp.zeros((L,), jnp.int32)
        for e in range(E):
            mi = (ev == e).astype(jnp.int32)
            rank = jnp.cumsum(mi) - 1
            dstv = dstv + mi * (off[e] + pre_run[e] + rank)
            pre_run[e] = pre_run[e] + jnp.sum(mi)
        dst_v[pl.ds(j * L, L)] = dstv
        a_iota = lax.broadcasted_iota(jnp.int32, (L,), 0)
        tok_v[pl.ds(j * L, L)] = (base + j * L + a_iota) // TOPK

    pltpu.sync_copy(dst_v, dst_hbm.at[pl.ds(base, CHUNK)])
    pltpu.async_copy(x_hbm.at[tok_v], rows_v, sem).wait()
    pltpu.async_copy(rows_v, xs_hbm.at[dst_v], sem).wait()

    @pl.when(wid == 0)
    def _():
        e_last = jnp.int32(0)
        for e in range(1, E):
            e_last = jnp.where(cnt[e] > 0, jnp.int32(e), e_last)
        tb = cumblk[E - 1]

        used = [cnt[e] > 0 for e in range(E)]
        rank = []
        r = jnp.int32(0)
        fb = []
        for e in range(E):
            rank.append(r)
            r = r + used[e].astype(jnp.int32)
            fb.append(off[e] // BM)
        start = []
        sprev = jnp.int32(-1)
        for e in range(E):
            fb2 = jnp.int32(0)
            for e2 in range(E):
                fb2 = jnp.where(
                    used[e2] & (rank[e2] == rank[e] - 2), fb[e2], fb2
                )
            s_e = jnp.where(
                rank[e] == 0,
                jnp.int32(0),
                jnp.where(rank[e] == 1, jnp.int32(1),
                          jnp.maximum(fb2, sprev + 1)),
            )
            start.append(s_e)
            sprev = jnp.where(used[e], s_e, sprev)

        def block_experts(bv):
            bev = jnp.zeros((L,), jnp.int32)
            for e in range(E - 1):
                bev = bev + (bv >= cumblk[e]).astype(jnp.int32)
            return jnp.minimum(bev, e_last)

        for k in range(4):
            bv = lax.broadcasted_iota(jnp.int32, (L,), 0) + k * L
            bev = jnp.where(bv == NB, tb, block_experts(bv))
            be_v[pl.ds(k * L, L)] = bev
        for k in range(4):
            bv = lax.broadcasted_iota(jnp.int32, (L,), 0) + k * L
            bev = block_experts(bv)
            us = jnp.zeros((L,), jnp.int32)
            for e in range(E):
                us = jnp.where(bev == e, rank[e] % 3, us)
            be_v[pl.ds((4 + k) * L, L)] = us
        for k in range(4):
            bv = lax.broadcasted_iota(jnp.int32, (L,), 0) + k * L
            fe = jnp.full((L,), -1, jnp.int32)
            fs = jnp.zeros((L,), jnp.int32)
            for e in range(E):
                hit = used[e] & (bv == start[e])
                fe = jnp.where(hit, jnp.int32(e), fe)
                fs = jnp.where(hit, rank[e] % 3, fs)
            be_v[pl.ds((8 + k) * L, L)] = fe
            be_v[pl.ds((12 + k) * L, L)] = fs
        pltpu.sync_copy(be_v, be_hbm)


def _dispatch(eflat, x2):
    return pl.kernel(
        _dispatch_body,
        mesh=plsc.VectorSubcoreMesh(core_axis_name="c", subcore_axis_name="s"),
        compiler_params=pltpu.CompilerParams(needs_layout_passes=False),
        out_type=[
            jax.ShapeDtypeStruct((NPAD, DIM), jnp.float32),
            jax.ShapeDtypeStruct((NA,), jnp.int32),
            jax.ShapeDtypeStruct((NBPAD,), jnp.int32),
        ],
        scratch_types=[
            pltpu.VMEM((NA,), jnp.int32),
            pltpu.VMEM((CHUNK,), jnp.int32),
            pltpu.VMEM((CHUNK,), jnp.int32),
            pltpu.VMEM((CHUNK, DIM), jnp.float32),
            pltpu.VMEM((NBPAD,), jnp.int32),
            pltpu.SemaphoreType.DMA,
        ],
    )(eflat, x2)


def _gemm_body(be_ref, xs_ref, w1_hbm, b1_ref, w2_hbm, b2_ref, ys_ref,
               w1b, w2b, s1a, s1b, s1c, s2a, s2b, s2c):
    i = pl.program_id(0)
    sems1 = [s1a, s1b, s1c]
    sems2 = [s2a, s2b, s2c]
    fe = be_ref[128 + i]
    fs = be_ref[192 + i]

    for s in range(3):
        @pl.when((fe >= 0) & (fs == s))
        def _(s=s):
            pltpu.make_async_copy(w1_hbm.at[fe], w1b.at[s], sems1[s]).start()
            pltpu.make_async_copy(w2_hbm.at[fe], w2b.at[s], sems2[s]).start()

    @pl.when(i < be_ref[NB])
    def _():
        e = be_ref[i]
        prev = be_ref[jnp.maximum(i - 1, 0)]
        first = jnp.logical_or(i == 0, e != prev)
        us = be_ref[64 + i]
        for s in range(3):
            @pl.when(first & (us == s))
            def _(s=s):
                pltpu.make_async_copy(w1_hbm.at[e], w1b.at[s], sems1[s]).wait()
                pltpu.make_async_copy(w2_hbm.at[e], w2b.at[s], sems2[s]).wait()

        h = jnp.maximum(
            jnp.dot(xs_ref[...], w1b[us], preferred_element_type=jnp.float32)
            + b1_ref[0],
            0.0,
        )
        ys_ref[...] = (
            jnp.dot(h, w2b[us], preferred_element_type=jnp.float32) + b2_ref[0]
        )


def _gemm(be, xs, W1, b1, W2, b2):
    grid_spec = pltpu.PrefetchScalarGridSpec(
        num_scalar_prefetch=1,
        grid=(NB,),
        in_specs=[
            pl.BlockSpec((BM, DIM), lambda i, be_ref: (i, 0)),
            pl.BlockSpec(memory_space=pl.ANY),
            pl.BlockSpec((1, 1, HID), lambda i, be_ref: (be_ref[i], 0, 0)),
            pl.BlockSpec(memory_space=pl.ANY),
            pl.BlockSpec((1, 1, DIM), lambda i, be_ref: (0, 0, 0)),
        ],
        out_specs=pl.BlockSpec((BM, DIM), lambda i, be_ref: (i, 0)),
        scratch_shapes=[
            pltpu.VMEM((3, DIM, HID), jnp.float32),
            pltpu.VMEM((3, HID, DIM), jnp.float32),
        ] + [pltpu.SemaphoreType.DMA] * 6,
    )
    return pl.pallas_call(
        _gemm_body,
        grid_spec=grid_spec,
        out_shape=jax.ShapeDtypeStruct((NPAD, DIM), jnp.float32),
        compiler_params=pltpu.CompilerParams(
            dimension_semantics=("arbitrary",),
            vmem_limit_bytes=100 * 1024 * 1024,
        ),
    )(be, xs, W1, b1.reshape(E, 1, HID), W2, b2.reshape(E, 1, DIM))


CB = 4
CBA = CHUNK // CB
CBT = TPW // CB


def _combine_body(ys_hbm, dst_hbm, w_hbm, y_hbm,
                  dst_v, w_v, rows_a, rows_b, out_v, sem_a, sem_b):
    cid = lax.axis_index("c")
    sid = lax.axis_index("s")
    wid = sid * 2 + cid
    abase = wid * CHUNK
    tbase = wid * TPW

    pltpu.sync_copy(dst_hbm.at[pl.ds(abase, CHUNK)], dst_v)
    pltpu.sync_copy(w_hbm.at[pl.ds(abase, CHUNK)], w_v)

    bufs = [(rows_a, sem_a), (rows_b, sem_b)]

    def gather(bi):
        rows, sem = bufs[bi % 2]
        return pltpu.async_copy(
            ys_hbm.at[dst_v.at[pl.ds(bi * CBA, CBA)]], rows, sem
        )

    pending = gather(0)
    for bi in range(CB):
        rows_v, _ = bufs[bi % 2]
        pending.wait()
        if bi + 1 < CB:
            pending = gather(bi + 1)

        def tok_body(t, _):
            wa = plsc.load_gather(w_v, [jnp.full((L,), bi * CBA + 2 * t, jnp.int32)])
            wb = plsc.load_gather(w_v, [jnp.full((L,), bi * CBA + 2 * t + 1, jnp.int32)])
            for j in range(DIM // L):
                a = rows_v[2 * t, pl.ds(j * L, L)]
                b = rows_v[2 * t + 1, pl.ds(j * L, L)]
                out_v[t, pl.ds(j * L, L)] = wa * a + wb * b
            return 0

        lax.fori_loop(0, CBT, tok_body, 0)
        pltpu.sync_copy(out_v, y_hbm.at[pl.ds(tbase + bi * CBT, CBT)])


def _combine(ys, dst, wflat):
    return pl.kernel(
        _combine_body,
        mesh=plsc.VectorSubcoreMesh(core_axis_name="c", subcore_axis_name="s"),
        compiler_params=pltpu.CompilerParams(needs_layout_passes=False),
        out_type=jax.ShapeDtypeStruct((S, DIM), jnp.float32),
        scratch_types=[
            pltpu.VMEM((CHUNK,), jnp.int32),
            pltpu.VMEM((CHUNK,), jnp.float32),
            pltpu.VMEM((CBA, DIM), jnp.float32),
            pltpu.VMEM((CBA, DIM), jnp.float32),
            pltpu.VMEM((CBT, DIM), jnp.float32),
            pltpu.SemaphoreType.DMA,
            pltpu.SemaphoreType.DMA,
        ],
    )(ys, dst, wflat)


@jax.jit
def kernel(x, Wg, W1, b1, W2, b2):
    x2 = x.reshape(S, DIM)
    eid, wt = _router(x2, Wg)
    xs, dst, be = _dispatch(eid.reshape(NA), x2)
    ys = _gemm(be, xs, W1, b1, W2, b2)
    y = _combine(ys, dst, wt.reshape(NA))
    return y.reshape(B, S, DIM)
